# baseline (speedup 1.0000x reference)
import functools

import jax
import jax.numpy as jnp
from jax import lax
from jax.experimental import pallas as pl
from jax.experimental.pallas import tpu as pltpu
from jax.experimental.pallas import tpu_sc as plsc

N = 10000
N_PAD = 10240
NROW = N_PAD // 128
E = 320000
IN_DIM = 128
HID = 256
H2 = HID // 2
NC, NS, L = 2, 16, 16
RB = 512
GRID = N_PAD // RB

EPT = E // NS
BSZ = 80
NBATCH = EPT // BSZ
RPT = N_PAD // NS
CCH = 32

EPW = E // (NC * NS)
DBS = 400
NDB = EPW // DBS

_mesh = plsc.VectorSubcoreMesh(
    core_axis_name="c", subcore_axis_name="s", num_cores=NC, num_subcores=NS)

_sc_params = pltpu.CompilerParams(use_tc_tiling_on_sc=False,
                                 needs_layout_passes=False)


def _zero_vmem_2d(ref, rows, cols):
  def body(r, _):
    for k in range(cols // L):
      ref[r, pl.ds(k * L, L)] = jnp.zeros((L,), jnp.float32)
    return 0
  lax.fori_loop(0, rows, body, 0)


def _iota_rows(ref, n):
  base = lax.iota(jnp.int32, L)
  for j in range(n // L):
    ref[pl.ds(j * L, L)] = base + (j * L)


@functools.partial(
    pl.kernel,
    out_type=jax.ShapeDtypeStruct((NC, NROW, 128), jnp.float32),
    mesh=_mesh,
    compiler_params=_sc_params,
    scratch_types=dict(
        dstv=pltpu.VMEM((DBS,), jnp.int32),
        ewv=pltpu.VMEM((DBS,), jnp.float32),
        deg_local=pltpu.VMEM((NROW, 128), jnp.float32),
        rowidx=pltpu.VMEM((NROW,), jnp.int32),
        zrow=pltpu.VMEM((NROW // NS, 128), jnp.float32),
        tmp=pltpu.VMEM((NROW // NS, 128), jnp.float32),
        acc=pltpu.VMEM_SHARED((NROW, 128), jnp.float32),
    ),
)
def _sc_deg(dst_hbm, ew_hbm, out_hbm, dstv, ewv, deg_local, rowidx, zrow,
            tmp, acc):
  cid = lax.axis_index("c")
  sid = lax.axis_index("s")
  rpt = NROW // NS

  _zero_vmem_2d(deg_local, NROW, 128)
  _zero_vmem_2d(zrow, rpt, 128)
  _iota_rows(rowidx, NROW)
  pltpu.sync_copy(zrow, acc.at[pl.ds(sid * rpt, rpt)])

  base = cid * (E // NC) + sid * EPW

  def batch(j, _):
    off = base + j * DBS
    pltpu.sync_copy(dst_hbm.at[pl.ds(off, DBS)], dstv)
    pltpu.sync_copy(ew_hbm.at[pl.ds(off, DBS)], ewv)

    def inner(k, _):
      idx = dstv[pl.ds(k * L, L)]
      w = ewv[pl.ds(k * L, L)]
      plsc.addupdate_scatter(deg_local, [idx >> 7, idx & 127], w)
      return 0

    lax.fori_loop(0, DBS // L, inner, 0)
    return 0

  lax.fori_loop(0, NDB, batch, 0)

  plsc.subcore_barrier()
  pltpu.sync_copy(deg_local, acc.at[rowidx], add=True)
  plsc.subcore_barrier()

  pltpu.sync_copy(acc.at[pl.ds(sid * rpt, rpt)], tmp)
  pltpu.sync_copy(tmp, out_hbm.at[cid, pl.ds(sid * rpt, rpt)])


NPAIR = NBATCH // 2


@functools.partial(
    pl.kernel,
    out_type=jax.ShapeDtypeStruct((NC, N_PAD, H2), jnp.float32),
    mesh=_mesh,
    compiler_params=_sc_params,
    scratch_types=dict(
        srcv=pltpu.VMEM((2, BSZ), jnp.int32),
        dsta=pltpu.VMEM((NBATCH, BSZ), jnp.int32),
        ewv=pltpu.VMEM((2, BSZ), jnp.float32),
        rows=pltpu.VMEM((2, BSZ, H2), jnp.float32),
        zbuf=pltpu.VMEM((CCH, H2), jnp.float32),
        acc=pltpu.VMEM_SHARED((N_PAD, H2), jnp.float32),
        semi0=pltpu.SemaphoreType.DMA,
        semi1=pltpu.SemaphoreType.DMA,
        semg0=pltpu.SemaphoreType.DMA,
        semg1=pltpu.SemaphoreType.DMA,
        sems0=pltpu.SemaphoreType.DMA,
        sems1=pltpu.SemaphoreType.DMA,
    ),
)
def _sc_agg(hwp_hbm, src_hbm, dst2_hbm, ew_hbm, out_hbm, srcv, dsta, ewv,
            rows, zbuf, acc, semi0, semi1, semg0, semg1, sems0, sems1):
  cid = lax.axis_index("c")
  sid = lax.axis_index("s")
  semi = (semi0, semi1)
  semg = (semg0, semg1)
  sems = (sems0, sems1)
  base = sid * EPT
  table = hwp_hbm.at[cid]

  def idx_start(j, b):
    off = base + j * BSZ
    pltpu.async_copy(src_hbm.at[pl.ds(off, BSZ)], srcv.at[b], semi[b])
    pltpu.async_copy(ew_hbm.at[pl.ds(off, BSZ)], ewv.at[b], semi[b])

  def idx_wait(j, b):
    off = base + j * BSZ
    pltpu.make_async_copy(src_hbm.at[pl.ds(off, BSZ)], srcv.at[b],
                          semi[b]).wait()
    pltpu.make_async_copy(ew_hbm.at[pl.ds(off, BSZ)], ewv.at[b],
                          semi[b]).wait()

  def gather_start(b):
    pltpu.async_copy(table.at[srcv.at[b]], rows.at[b], semg[b])

  def gather_wait(b):
    pltpu.make_async_copy(table.at[srcv.at[b]], rows.at[b], semg[b]).wait()

  def scat_start(b, j):
    pltpu.async_copy(rows.at[b], acc.at[dsta.at[j]], sems[b], add=True)

  def scat_wait(b, j):
    pltpu.make_async_copy(rows.at[b], acc.at[dsta.at[j]], sems[b]).wait()

  def scale(b):
    @plsc.parallel_loop(0, BSZ, unroll=4)
    def _(e):
      ew16 = plsc.load_gather(ewv.at[b], [jnp.full((L,), e, jnp.int32)])
      for k in range(H2 // L):
        rows[b, e, pl.ds(k * L, L)] = rows[b, e, pl.ds(k * L, L)] * ew16

  pltpu.async_copy(dst2_hbm.at[pl.ds(sid * NBATCH, NBATCH)], dsta, semg1)
  idx_start(0, 0)
  idx_start(1, 1)

  _zero_vmem_2d(zbuf, CCH, H2)
  for k in range(RPT // CCH):
    pltpu.sync_copy(zbuf, acc.at[pl.ds(sid * RPT + k * CCH, CCH)])
  plsc.subcore_barrier()

  pltpu.make_async_copy(dst2_hbm.at[pl.ds(sid * NBATCH, NBATCH)], dsta,
                        semg1).wait()
  idx_wait(0, 0)
  gather_start(0)

  def pair(t, _):
    j0 = 2 * t
    j1 = j0 + 1
    more = t < (NPAIR - 1)
    idx_wait(j1, 1)

    @pl.when(t >= 1)
    def _():
      scat_wait(1, j1 - 2)

    gather_start(1)
    gather_wait(0)
    scale(0)

    @pl.when(more)
    def _():
      idx_start(j0 + 2, 0)

    scat_start(0, j0)
    gather_wait(1)
    scale(1)

    @pl.when(more)
    def _():
      idx_start(j1 + 2, 1)

    scat_start(1, j1)

    @pl.when(more)
    def _():
      idx_wait(j0 + 2, 0)
      scat_wait(0, j0)
      gather_start(0)
    return 0

  lax.fori_loop(0, NPAIR, pair, 0)
  scat_wait(0, 2 * NPAIR - 2)
  scat_wait(1, 2 * NPAIR - 1)
  plsc.subcore_barrier()

  for k in range(RPT // CCH):
    r0 = sid * RPT + k * CCH
    pltpu.sync_copy(acc.at[pl.ds(r0, CCH)], zbuf)
    pltpu.sync_copy(zbuf, out_hbm.at[cid, pl.ds(r0, CCH)])


def _dinv_body(degp_ref, out_ref):
  deg = degp_ref[0] + degp_ref[1] + 2.0
  out_ref[...] = jnp.where(deg > 0, lax.rsqrt(jnp.where(deg > 0, deg, 1.0)),
                           0.0)


def _tc_dinv(degp):
  return pl.pallas_call(
      _dinv_body,
      out_shape=jax.ShapeDtypeStruct((NROW, 128), jnp.float32),
  )(degp)


def _mm_body(x_ref, w_ref, dinv_ref, out_ref):
  r = jnp.dot(x_ref[...], w_ref[...], preferred_element_type=jnp.float32)
  r = r * dinv_ref[...]
  out_ref[0] = r[:, :H2]
  out_ref[1] = r[:, H2:]


def _tc_mm(x, w, dinv):
  d = x.shape[1]
  return pl.pallas_call(
      _mm_body,
      grid=(GRID,),
      in_specs=[
          pl.BlockSpec((RB, d), lambda i: (i, 0)),
          pl.BlockSpec((d, HID), lambda i: (0, 0)),
          pl.BlockSpec((RB, 1), lambda i: (i, 0)),
      ],
      out_specs=pl.BlockSpec((NC, RB, H2), lambda i: (0, i, 0)),
      out_shape=jax.ShapeDtypeStruct((NC, N_PAD, H2), jnp.float32),
  )(x, w, dinv)


def _post_body(agg_ref, hwp_ref, dinv_ref, b_ref, g_ref, be_ref, out_ref, *,
               relu):
  agg = jnp.concatenate([agg_ref[0], agg_ref[1]], axis=1)
  hwp = jnp.concatenate([hwp_ref[0], hwp_ref[1]], axis=1)
  dinv = dinv_ref[...]
  h = dinv * agg + 2.0 * dinv * hwp + b_ref[...]
  mu = jnp.mean(h, axis=-1, keepdims=True)
  var = jnp.mean((h - mu) * (h - mu), axis=-1, keepdims=True)
  h = (h - mu) * lax.rsqrt(var + 1e-5) * g_ref[...] + be_ref[...]
  if relu:
    h = jnp.maximum(h, 0.0)
  out_ref[...] = h


def _tc_post(agg, hwp, dinv, b, g, be, relu):
  return pl.pallas_call(
      functools.partial(_post_body, relu=relu),
      grid=(GRID,),
      in_specs=[
          pl.BlockSpec((NC, RB, H2), lambda i: (0, i, 0)),
          pl.BlockSpec((NC, RB, H2), lambda i: (0, i, 0)),
          pl.BlockSpec((RB, 1), lambda i: (i, 0)),
          pl.BlockSpec((1, HID), lambda i: (0, 0)),
          pl.BlockSpec((1, HID), lambda i: (0, 0)),
          pl.BlockSpec((1, HID), lambda i: (0, 0)),
      ],
      out_specs=pl.BlockSpec((RB, HID), lambda i: (i, 0)),
      out_shape=jax.ShapeDtypeStruct((N_PAD, HID), jnp.float32),
  )(agg, hwp, dinv, b, g, be)


def _postmm_body(agg_ref, hwp_ref, dinv_ref, b_ref, g_ref, be_ref, w_ref,
                 out_ref):
  agg = jnp.concatenate([agg_ref[0], agg_ref[1]], axis=1)
  hwp = jnp.concatenate([hwp_ref[0], hwp_ref[1]], axis=1)
  dinv = dinv_ref[...]
  h = dinv * agg + 2.0 * dinv * hwp + b_ref[...]
  mu = jnp.mean(h, axis=-1, keepdims=True)
  var = jnp.mean((h - mu) * (h - mu), axis=-1, keepdims=True)
  h = (h - mu) * lax.rsqrt(var + 1e-5) * g_ref[...] + be_ref[...]
  h = jnp.maximum(h, 0.0)
  r = jnp.dot(h, w_ref[...], preferred_element_type=jnp.float32) * dinv
  out_ref[0] = r[:, :H2]
  out_ref[1] = r[:, H2:]


def _tc_postmm(agg, hwp, dinv, b, g, be, w):
  return pl.pallas_call(
      _postmm_body,
      grid=(GRID,),
      in_specs=[
          pl.BlockSpec((NC, RB, H2), lambda i: (0, i, 0)),
          pl.BlockSpec((NC, RB, H2), lambda i: (0, i, 0)),
          pl.BlockSpec((RB, 1), lambda i: (i, 0)),
          pl.BlockSpec((1, HID), lambda i: (0, 0)),
          pl.BlockSpec((1, HID), lambda i: (0, 0)),
          pl.BlockSpec((1, HID), lambda i: (0, 0)),
          pl.BlockSpec((HID, HID), lambda i: (0, 0)),
      ],
      out_specs=pl.BlockSpec((NC, RB, H2), lambda i: (0, i, 0)),
      out_shape=jax.ShapeDtypeStruct((NC, N_PAD, H2), jnp.float32),
  )(agg, hwp, dinv, b, g, be, w)


def _tail_body(agg_ref, hwp_ref, dinv_ref, b2_ref, g2_ref, be2_ref, x_ref,
               wih_ref, bih_ref, bhh_ref, wd1_ref, bd1_ref, wd2_ref, bd2_ref,
               wd3_ref, bd3_ref, out_ref):
  agg = jnp.concatenate([agg_ref[0], agg_ref[1]], axis=1)
  hwp = jnp.concatenate([hwp_ref[0], hwp_ref[1]], axis=1)
  dinv = dinv_ref[...]
  h = dinv * agg + 2.0 * dinv * hwp + b2_ref[...]
  mu = jnp.mean(h, axis=-1, keepdims=True)
  var = jnp.mean((h - mu) * (h - mu), axis=-1, keepdims=True)
  h = (h - mu) * lax.rsqrt(var + 1e-5) * g2_ref[...] + be2_ref[...]
  gi = jnp.dot(h, wih_ref[...], preferred_element_type=jnp.float32) + \
      bih_ref[...]
  bhh = bhh_ref[...]
  r = jax.nn.sigmoid(gi[:, :HID] + bhh[:, :HID])
  z = jax.nn.sigmoid(gi[:, HID:2 * HID] + bhh[:, HID:2 * HID])
  n = jnp.tanh(gi[:, 2 * HID:] + r * bhh[:, 2 * HID:])
  temporal = (1.0 - z) * n
  d1 = jnp.dot(temporal, wd1_ref[:HID, :],
               preferred_element_type=jnp.float32)
  d1 = d1 + jnp.dot(x_ref[...], wd1_ref[HID:, :],
                    preferred_element_type=jnp.float32)
  d1 = jnp.maximum(d1 + bd1_ref[...], 0.0)
  d2 = jnp.maximum(
      jnp.dot(d1, wd2_ref[...], preferred_element_type=jnp.float32) +
      bd2_ref[...], 0.0)
  pred = jnp.dot(d2, wd3_ref[...], preferred_element_type=jnp.float32) + \
      bd3_ref[...]
  out_ref[...] = jnp.clip(pred, -5.0, 5.0)


def _tc_tail(agg, hwp, dinv, b2, g2, be2, x, wih_t, b_ih, b_hh, wd1, bd1,
             wd2, bd2, wd3p, bd3p):
  return pl.pallas_call(
      _tail_body,
      grid=(GRID,),
      in_specs=[
          pl.BlockSpec((NC, RB, H2), lambda i: (0, i, 0)),
          pl.BlockSpec((NC, RB, H2), lambda i: (0, i, 0)),
          pl.BlockSpec((RB, 1), lambda i: (i, 0)),
          pl.BlockSpec((1, HID), lambda i: (0, 0)),
          pl.BlockSpec((1, HID), lambda i: (0, 0)),
          pl.BlockSpec((1, HID), lambda i: (0, 0)),
          pl.BlockSpec((RB, IN_DIM), lambda i: (i, 0)),
          pl.BlockSpec((HID, 3 * HID), lambda i: (0, 0)),
          pl.BlockSpec((1, 3 * HID), lambda i: (0, 0)),
          pl.BlockSpec((1, 3 * HID), lambda i: (0, 0)),
          pl.BlockSpec((HID + IN_DIM, HID), lambda i: (0, 0)),
          pl.BlockSpec((1, HID), lambda i: (0, 0)),
          pl.BlockSpec((HID, H2), lambda i: (0, 0)),
          pl.BlockSpec((1, H2), lambda i: (0, 0)),
          pl.BlockSpec((H2, 128), lambda i: (0, 0)),
          pl.BlockSpec((1, 128), lambda i: (0, 0)),
      ],
      out_specs=pl.BlockSpec((RB, 128), lambda i: (i, 0)),
      out_shape=jax.ShapeDtypeStruct((N_PAD, 128), jnp.float32),
  )(agg, hwp, dinv, b2, g2, be2, x, wih_t, b_ih, b_hh, wd1, bd1, wd2, bd2,
    wd3p, bd3p)


@jax.jit
def kernel(x, edge_index, edge_weight, W1, b1, g1, be1, W2, b2, g2, be2,
           W_ih, W_hh, b_ih, b_hh, Wd1, bd1, Wd2, bd2, Wd3, bd3):
  src = edge_index[0]
  dst = edge_index[1]
  x_pad = jnp.pad(x, ((0, N_PAD - N), (0, 0)))

  dst2 = dst.reshape(E // BSZ, BSZ)
  degp = _sc_deg(dst, edge_weight)
  dinv = _tc_dinv(degp).reshape(N_PAD, 1)

  hwp1 = _tc_mm(x_pad, W1, dinv)
  agg1 = _sc_agg(hwp1, src, dst2, edge_weight)
  hwp2 = _tc_postmm(agg1, hwp1, dinv, b1.reshape(1, HID),
                    g1.reshape(1, HID), be1.reshape(1, HID), W2)
  agg2 = _sc_agg(hwp2, src, dst2, edge_weight)

  pred_pad = _tc_tail(
      agg2, hwp2, dinv, b2.reshape(1, HID), g2.reshape(1, HID),
      be2.reshape(1, HID), x_pad, W_ih.T, b_ih.reshape(1, 3 * HID),
      b_hh.reshape(1, 3 * HID), Wd1, bd1.reshape(1, HID), Wd2,
      bd2.reshape(1, H2),
      jnp.pad(Wd3, ((0, 0), (0, 128 - Wd3.shape[1]))),
      jnp.pad(bd3, (0, 128 - bd3.shape[0])).reshape(1, 128))
  return pred_pad[:N, :2]


# speedup vs baseline: 1.0053x; 1.0053x over previous
import functools

import jax
import jax.numpy as jnp
from jax import lax
from jax.experimental import pallas as pl
from jax.experimental.pallas import tpu as pltpu
from jax.experimental.pallas import tpu_sc as plsc

N = 10000
N_PAD = 10240
NROW = N_PAD // 128
E = 320000
IN_DIM = 128
HID = 256
H2 = HID // 2
NC, NS, L = 2, 16, 16
RB = 512
GRID = N_PAD // RB

EPT = E // NS
BSZ = 80
NBATCH = EPT // BSZ
RPT = N_PAD // NS
CCH = 32

EPW = E // (NC * NS)
DBS = 400
NDB = EPW // DBS

_mesh = plsc.VectorSubcoreMesh(
    core_axis_name="c", subcore_axis_name="s", num_cores=NC, num_subcores=NS)

_sc_params = pltpu.CompilerParams(use_tc_tiling_on_sc=False,
                                 needs_layout_passes=False)


def _zero_vmem_2d(ref, rows, cols):
  def body(r, _):
    for k in range(cols // L):
      ref[r, pl.ds(k * L, L)] = jnp.zeros((L,), jnp.float32)
    return 0
  lax.fori_loop(0, rows, body, 0)


def _iota_rows(ref, n):
  base = lax.iota(jnp.int32, L)
  for j in range(n // L):
    ref[pl.ds(j * L, L)] = base + (j * L)


@functools.partial(
    pl.kernel,
    out_type=jax.ShapeDtypeStruct((NC, NROW, 128), jnp.float32),
    mesh=_mesh,
    compiler_params=_sc_params,
    scratch_types=dict(
        dstv=pltpu.VMEM((DBS,), jnp.int32),
        ewv=pltpu.VMEM((DBS,), jnp.float32),
        deg_local=pltpu.VMEM((NROW, 128), jnp.float32),
        rowidx=pltpu.VMEM((NROW,), jnp.int32),
        zrow=pltpu.VMEM((NROW // NS, 128), jnp.float32),
        tmp=pltpu.VMEM((NROW // NS, 128), jnp.float32),
        acc=pltpu.VMEM_SHARED((NROW, 128), jnp.float32),
    ),
)
def _sc_deg(dst_hbm, ew_hbm, out_hbm, dstv, ewv, deg_local, rowidx, zrow,
            tmp, acc):
  cid = lax.axis_index("c")
  sid = lax.axis_index("s")
  rpt = NROW // NS

  _zero_vmem_2d(deg_local, NROW, 128)
  _zero_vmem_2d(zrow, rpt, 128)
  _iota_rows(rowidx, NROW)
  pltpu.sync_copy(zrow, acc.at[pl.ds(sid * rpt, rpt)])

  base = cid * (E // NC) + sid * EPW

  def batch(j, _):
    off = base + j * DBS
    pltpu.sync_copy(dst_hbm.at[pl.ds(off, DBS)], dstv)
    pltpu.sync_copy(ew_hbm.at[pl.ds(off, DBS)], ewv)

    def inner(k, _):
      idx = dstv[pl.ds(k * L, L)]
      w = ewv[pl.ds(k * L, L)]
      plsc.addupdate_scatter(deg_local, [idx >> 7, idx & 127], w)
      return 0

    lax.fori_loop(0, DBS // L, inner, 0)
    return 0

  lax.fori_loop(0, NDB, batch, 0)

  plsc.subcore_barrier()
  pltpu.sync_copy(deg_local, acc.at[rowidx], add=True)
  plsc.subcore_barrier()

  pltpu.sync_copy(acc.at[pl.ds(sid * rpt, rpt)],
                  out_hbm.at[cid, pl.ds(sid * rpt, rpt)])


NPAIR = NBATCH // 2


@functools.partial(
    pl.kernel,
    out_type=jax.ShapeDtypeStruct((NC, N_PAD, H2), jnp.float32),
    mesh=_mesh,
    compiler_params=_sc_params,
    scratch_types=dict(
        srcv=pltpu.VMEM((2, BSZ), jnp.int32),
        dsta=pltpu.VMEM((NBATCH, BSZ), jnp.int32),
        ewv=pltpu.VMEM((2, BSZ), jnp.float32),
        rows=pltpu.VMEM((2, BSZ, H2), jnp.float32),
        zbuf=pltpu.VMEM((CCH, H2), jnp.float32),
        acc=pltpu.VMEM_SHARED((N_PAD, H2), jnp.float32),
        semi0=pltpu.SemaphoreType.DMA,
        semi1=pltpu.SemaphoreType.DMA,
        semg0=pltpu.SemaphoreType.DMA,
        semg1=pltpu.SemaphoreType.DMA,
        sems0=pltpu.SemaphoreType.DMA,
        sems1=pltpu.SemaphoreType.DMA,
    ),
)
def _sc_agg(hwp_hbm, src_hbm, dst2_hbm, ew_hbm, out_hbm, srcv, dsta, ewv,
            rows, zbuf, acc, semi0, semi1, semg0, semg1, sems0, sems1):
  cid = lax.axis_index("c")
  sid = lax.axis_index("s")
  semi = (semi0, semi1)
  semg = (semg0, semg1)
  sems = (sems0, sems1)
  base = sid * EPT
  table = hwp_hbm.at[cid]

  def idx_start(j, b):
    off = base + j * BSZ
    pltpu.async_copy(src_hbm.at[pl.ds(off, BSZ)], srcv.at[b], semi[b])
    pltpu.async_copy(ew_hbm.at[pl.ds(off, BSZ)], ewv.at[b], semi[b])

  def idx_wait(j, b):
    off = base + j * BSZ
    pltpu.make_async_copy(src_hbm.at[pl.ds(off, BSZ)], srcv.at[b],
                          semi[b]).wait()
    pltpu.make_async_copy(ew_hbm.at[pl.ds(off, BSZ)], ewv.at[b],
                          semi[b]).wait()

  def gather_start(b):
    pltpu.async_copy(table.at[srcv.at[b]], rows.at[b], semg[b])

  def gather_wait(b):
    pltpu.make_async_copy(table.at[srcv.at[b]], rows.at[b], semg[b]).wait()

  def scat_start(b, j):
    pltpu.async_copy(rows.at[b], acc.at[dsta.at[j]], sems[b], add=True)

  def scat_wait(b, j):
    pltpu.make_async_copy(rows.at[b], acc.at[dsta.at[j]], sems[b]).wait()

  def scale(b):
    @plsc.parallel_loop(0, BSZ, unroll=8)
    def _(e):
      ew16 = plsc.load_gather(ewv.at[b], [jnp.full((L,), e, jnp.int32)])
      for k in range(H2 // L):
        rows[b, e, pl.ds(k * L, L)] = rows[b, e, pl.ds(k * L, L)] * ew16

  pltpu.async_copy(dst2_hbm.at[pl.ds(sid * NBATCH, NBATCH)], dsta, semg1)
  idx_start(0, 0)
  idx_start(1, 1)

  _zero_vmem_2d(zbuf, CCH, H2)
  for k in range(RPT // CCH):
    pltpu.sync_copy(zbuf, acc.at[pl.ds(sid * RPT + k * CCH, CCH)])
  plsc.subcore_barrier()

  pltpu.make_async_copy(dst2_hbm.at[pl.ds(sid * NBATCH, NBATCH)], dsta,
                        semg1).wait()
  idx_wait(0, 0)
  gather_start(0)

  def pair(t, _):
    j0 = 2 * t
    j1 = j0 + 1
    more = t < (NPAIR - 1)
    idx_wait(j1, 1)

    @pl.when(t >= 1)
    def _():
      scat_wait(1, j1 - 2)

    gather_start(1)
    gather_wait(0)
    scale(0)

    @pl.when(more)
    def _():
      idx_start(j0 + 2, 0)

    scat_start(0, j0)
    gather_wait(1)
    scale(1)

    @pl.when(more)
    def _():
      idx_start(j1 + 2, 1)

    scat_start(1, j1)

    @pl.when(more)
    def _():
      idx_wait(j0 + 2, 0)
      scat_wait(0, j0)
      gather_start(0)
    return 0

  lax.fori_loop(0, NPAIR, pair, 0)
  scat_wait(0, 2 * NPAIR - 2)
  scat_wait(1, 2 * NPAIR - 1)
  plsc.subcore_barrier()

  pltpu.sync_copy(acc.at[pl.ds(sid * RPT, RPT)],
                  out_hbm.at[cid, pl.ds(sid * RPT, RPT)])


def _dinv_body(degp_ref, out_ref):
  deg = degp_ref[0] + degp_ref[1] + 2.0
  out_ref[...] = jnp.where(deg > 0, lax.rsqrt(jnp.where(deg > 0, deg, 1.0)),
                           0.0)


def _tc_dinv(degp):
  return pl.pallas_call(
      _dinv_body,
      out_shape=jax.ShapeDtypeStruct((NROW, 128), jnp.float32),
  )(degp)


def _mm_body(x_ref, w_ref, dinv_ref, out_ref):
  r = jnp.dot(x_ref[...], w_ref[...], preferred_element_type=jnp.float32)
  r = r * dinv_ref[...]
  out_ref[0] = r[:, :H2]
  out_ref[1] = r[:, H2:]


def _tc_mm(x, w, dinv):
  d = x.shape[1]
  return pl.pallas_call(
      _mm_body,
      grid=(GRID,),
      in_specs=[
          pl.BlockSpec((RB, d), lambda i: (i, 0)),
          pl.BlockSpec((d, HID), lambda i: (0, 0)),
          pl.BlockSpec((RB, 1), lambda i: (i, 0)),
      ],
      out_specs=pl.BlockSpec((NC, RB, H2), lambda i: (0, i, 0)),
      out_shape=jax.ShapeDtypeStruct((NC, N_PAD, H2), jnp.float32),
  )(x, w, dinv)


def _post_body(agg_ref, hwp_ref, dinv_ref, b_ref, g_ref, be_ref, out_ref, *,
               relu):
  agg = jnp.concatenate([agg_ref[0], agg_ref[1]], axis=1)
  hwp = jnp.concatenate([hwp_ref[0], hwp_ref[1]], axis=1)
  dinv = dinv_ref[...]
  h = dinv * agg + 2.0 * dinv * hwp + b_ref[...]
  mu = jnp.mean(h, axis=-1, keepdims=True)
  var = jnp.mean((h - mu) * (h - mu), axis=-1, keepdims=True)
  h = (h - mu) * lax.rsqrt(var + 1e-5) * g_ref[...] + be_ref[...]
  if relu:
    h = jnp.maximum(h, 0.0)
  out_ref[...] = h


def _tc_post(agg, hwp, dinv, b, g, be, relu):
  return pl.pallas_call(
      functools.partial(_post_body, relu=relu),
      grid=(GRID,),
      in_specs=[
          pl.BlockSpec((NC, RB, H2), lambda i: (0, i, 0)),
          pl.BlockSpec((NC, RB, H2), lambda i: (0, i, 0)),
          pl.BlockSpec((RB, 1), lambda i: (i, 0)),
          pl.BlockSpec((1, HID), lambda i: (0, 0)),
          pl.BlockSpec((1, HID), lambda i: (0, 0)),
          pl.BlockSpec((1, HID), lambda i: (0, 0)),
      ],
      out_specs=pl.BlockSpec((RB, HID), lambda i: (i, 0)),
      out_shape=jax.ShapeDtypeStruct((N_PAD, HID), jnp.float32),
  )(agg, hwp, dinv, b, g, be)


def _postmm_body(agg_ref, hwp_ref, dinv_ref, b_ref, g_ref, be_ref, w_ref,
                 out_ref):
  agg = jnp.concatenate([agg_ref[0], agg_ref[1]], axis=1)
  hwp = jnp.concatenate([hwp_ref[0], hwp_ref[1]], axis=1)
  dinv = dinv_ref[...]
  h = dinv * agg + 2.0 * dinv * hwp + b_ref[...]
  mu = jnp.mean(h, axis=-1, keepdims=True)
  var = jnp.mean((h - mu) * (h - mu), axis=-1, keepdims=True)
  h = (h - mu) * lax.rsqrt(var + 1e-5) * g_ref[...] + be_ref[...]
  h = jnp.maximum(h, 0.0)
  r = jnp.dot(h, w_ref[...], preferred_element_type=jnp.float32) * dinv
  out_ref[0] = r[:, :H2]
  out_ref[1] = r[:, H2:]


def _tc_postmm(agg, hwp, dinv, b, g, be, w):
  return pl.pallas_call(
      _postmm_body,
      grid=(GRID,),
      in_specs=[
          pl.BlockSpec((NC, RB, H2), lambda i: (0, i, 0)),
          pl.BlockSpec((NC, RB, H2), lambda i: (0, i, 0)),
          pl.BlockSpec((RB, 1), lambda i: (i, 0)),
          pl.BlockSpec((1, HID), lambda i: (0, 0)),
          pl.BlockSpec((1, HID), lambda i: (0, 0)),
          pl.BlockSpec((1, HID), lambda i: (0, 0)),
          pl.BlockSpec((HID, HID), lambda i: (0, 0)),
      ],
      out_specs=pl.BlockSpec((NC, RB, H2), lambda i: (0, i, 0)),
      out_shape=jax.ShapeDtypeStruct((NC, N_PAD, H2), jnp.float32),
  )(agg, hwp, dinv, b, g, be, w)


def _tail_body(agg_ref, hwp_ref, dinv_ref, b2_ref, g2_ref, be2_ref, x_ref,
               wih_ref, bih_ref, bhh_ref, wd1_ref, bd1_ref, wd2_ref, bd2_ref,
               wd3_ref, bd3_ref, out_ref):
  agg = jnp.concatenate([agg_ref[0], agg_ref[1]], axis=1)
  hwp = jnp.concatenate([hwp_ref[0], hwp_ref[1]], axis=1)
  dinv = dinv_ref[...]
  h = dinv * agg + 2.0 * dinv * hwp + b2_ref[...]
  mu = jnp.mean(h, axis=-1, keepdims=True)
  var = jnp.mean((h - mu) * (h - mu), axis=-1, keepdims=True)
  h = (h - mu) * lax.rsqrt(var + 1e-5) * g2_ref[...] + be2_ref[...]
  gi = jnp.dot(h, wih_ref[...], preferred_element_type=jnp.float32) + \
      bih_ref[...]
  bhh = bhh_ref[...]
  r = jax.nn.sigmoid(gi[:, :HID] + bhh[:, :HID])
  z = jax.nn.sigmoid(gi[:, HID:2 * HID] + bhh[:, HID:2 * HID])
  n = jnp.tanh(gi[:, 2 * HID:] + r * bhh[:, 2 * HID:])
  temporal = (1.0 - z) * n
  d1 = jnp.dot(temporal, wd1_ref[:HID, :],
               preferred_element_type=jnp.float32)
  d1 = d1 + jnp.dot(x_ref[...], wd1_ref[HID:, :],
                    preferred_element_type=jnp.float32)
  d1 = jnp.maximum(d1 + bd1_ref[...], 0.0)
  d2 = jnp.maximum(
      jnp.dot(d1, wd2_ref[...], preferred_element_type=jnp.float32) +
      bd2_ref[...], 0.0)
  pred = jnp.dot(d2, wd3_ref[...], preferred_element_type=jnp.float32) + \
      bd3_ref[...]
  out_ref[...] = jnp.clip(pred, -5.0, 5.0)


def _tc_tail(agg, hwp, dinv, b2, g2, be2, x, wih_t, b_ih, b_hh, wd1, bd1,
             wd2, bd2, wd3p, bd3p):
  return pl.pallas_call(
      _tail_body,
      grid=(GRID,),
      in_specs=[
          pl.BlockSpec((NC, RB, H2), lambda i: (0, i, 0)),
          pl.BlockSpec((NC, RB, H2), lambda i: (0, i, 0)),
          pl.BlockSpec((RB, 1), lambda i: (i, 0)),
          pl.BlockSpec((1, HID), lambda i: (0, 0)),
          pl.BlockSpec((1, HID), lambda i: (0, 0)),
          pl.BlockSpec((1, HID), lambda i: (0, 0)),
          pl.BlockSpec((RB, IN_DIM), lambda i: (i, 0)),
          pl.BlockSpec((HID, 3 * HID), lambda i: (0, 0)),
          pl.BlockSpec((1, 3 * HID), lambda i: (0, 0)),
          pl.BlockSpec((1, 3 * HID), lambda i: (0, 0)),
          pl.BlockSpec((HID + IN_DIM, HID), lambda i: (0, 0)),
          pl.BlockSpec((1, HID), lambda i: (0, 0)),
          pl.BlockSpec((HID, H2), lambda i: (0, 0)),
          pl.BlockSpec((1, H2), lambda i: (0, 0)),
          pl.BlockSpec((H2, 128), lambda i: (0, 0)),
          pl.BlockSpec((1, 128), lambda i: (0, 0)),
      ],
      out_specs=pl.BlockSpec((RB, 128), lambda i: (i, 0)),
      out_shape=jax.ShapeDtypeStruct((N_PAD, 128), jnp.float32),
  )(agg, hwp, dinv, b2, g2, be2, x, wih_t, b_ih, b_hh, wd1, bd1, wd2, bd2,
    wd3p, bd3p)


@jax.jit
def kernel(x, edge_index, edge_weight, W1, b1, g1, be1, W2, b2, g2, be2,
           W_ih, W_hh, b_ih, b_hh, Wd1, bd1, Wd2, bd2, Wd3, bd3):
  src = edge_index[0]
  dst = edge_index[1]
  x_pad = jnp.pad(x, ((0, N_PAD - N), (0, 0)))

  dst2 = dst.reshape(E // BSZ, BSZ)
  degp = _sc_deg(dst, edge_weight)
  dinv = _tc_dinv(degp).reshape(N_PAD, 1)

  hwp1 = _tc_mm(x_pad, W1, dinv)
  agg1 = _sc_agg(hwp1, src, dst2, edge_weight)
  hwp2 = _tc_postmm(agg1, hwp1, dinv, b1.reshape(1, HID),
                    g1.reshape(1, HID), be1.reshape(1, HID), W2)
  agg2 = _sc_agg(hwp2, src, dst2, edge_weight)

  pred_pad = _tc_tail(
      agg2, hwp2, dinv, b2.reshape(1, HID), g2.reshape(1, HID),
      be2.reshape(1, HID), x_pad, W_ih.T, b_ih.reshape(1, 3 * HID),
      b_hh.reshape(1, 3 * HID), Wd1, bd1.reshape(1, HID), Wd2,
      bd2.reshape(1, H2),
      jnp.pad(Wd3, ((0, 0), (0, 128 - Wd3.shape[1]))),
      jnp.pad(bd3, (0, 128 - bd3.shape[0])).reshape(1, 128))
  return pred_pad[:N, :2]


# speedup vs baseline: 1.0923x; 1.0865x over previous
import functools

import jax
import jax.numpy as jnp
from jax import lax
from jax.experimental import pallas as pl
from jax.experimental.pallas import tpu as pltpu
from jax.experimental.pallas import tpu_sc as plsc

N = 10000
N_PAD = 10240
NROW = N_PAD // 128
E = 320000
IN_DIM = 128
HID = 256
H2 = HID // 2
NC, NS, L = 2, 16, 16
RB = 512
GRID = N_PAD // RB

EPT = E // NS
BSZ = 80
NBATCH = EPT // BSZ
RPT = N_PAD // NS
CCH = 32

EPW = E // (NC * NS)
DBS = 400
NDB = EPW // DBS

_mesh = plsc.VectorSubcoreMesh(
    core_axis_name="c", subcore_axis_name="s", num_cores=NC, num_subcores=NS)

_sc_params = pltpu.CompilerParams(use_tc_tiling_on_sc=False,
                                 needs_layout_passes=False)


def _zero_vmem_2d(ref, rows, cols):
  def body(r, _):
    for k in range(cols // L):
      ref[r, pl.ds(k * L, L)] = jnp.zeros((L,), jnp.float32)
    return 0
  lax.fori_loop(0, rows, body, 0)


def _iota_rows(ref, n):
  base = lax.iota(jnp.int32, L)
  for j in range(n // L):
    ref[pl.ds(j * L, L)] = base + (j * L)


@functools.partial(
    pl.kernel,
    out_type=jax.ShapeDtypeStruct((NC, NROW, 128), jnp.float32),
    mesh=_mesh,
    compiler_params=_sc_params,
    scratch_types=dict(
        dstv=pltpu.VMEM((DBS,), jnp.int32),
        ewv=pltpu.VMEM((DBS,), jnp.float32),
        deg_local=pltpu.VMEM((NROW, 128), jnp.float32),
        rowidx=pltpu.VMEM((NROW,), jnp.int32),
        zrow=pltpu.VMEM((NROW // NS, 128), jnp.float32),
        tmp=pltpu.VMEM((NROW // NS, 128), jnp.float32),
        acc=pltpu.VMEM_SHARED((NROW, 128), jnp.float32),
    ),
)
def _sc_deg(dst_hbm, ew_hbm, out_hbm, dstv, ewv, deg_local, rowidx, zrow,
            tmp, acc):
  cid = lax.axis_index("c")
  sid = lax.axis_index("s")
  rpt = NROW // NS

  _zero_vmem_2d(deg_local, NROW, 128)
  _zero_vmem_2d(zrow, rpt, 128)
  _iota_rows(rowidx, NROW)
  pltpu.sync_copy(zrow, acc.at[pl.ds(sid * rpt, rpt)])

  base = cid * (E // NC) + sid * EPW

  def batch(j, _):
    off = base + j * DBS
    pltpu.sync_copy(dst_hbm.at[pl.ds(off, DBS)], dstv)
    pltpu.sync_copy(ew_hbm.at[pl.ds(off, DBS)], ewv)

    def inner(k, _):
      idx = dstv[pl.ds(k * L, L)]
      w = ewv[pl.ds(k * L, L)]
      plsc.addupdate_scatter(deg_local, [idx >> 7, idx & 127], w)
      return 0

    lax.fori_loop(0, DBS // L, inner, 0)
    return 0

  lax.fori_loop(0, NDB, batch, 0)

  plsc.subcore_barrier()
  pltpu.sync_copy(deg_local, acc.at[rowidx], add=True)
  plsc.subcore_barrier()

  pltpu.sync_copy(acc.at[pl.ds(sid * rpt, rpt)],
                  out_hbm.at[cid, pl.ds(sid * rpt, rpt)])


NTRI = (NBATCH - 1) // 3


@functools.partial(
    pl.kernel,
    out_type=jax.ShapeDtypeStruct((NC, N_PAD, H2), jnp.float32),
    mesh=_mesh,
    compiler_params=_sc_params,
    scratch_types=dict(
        srcv=pltpu.VMEM((3, BSZ), jnp.int32),
        dstv=pltpu.VMEM((4, BSZ), jnp.int32),
        ewv=pltpu.VMEM((3, BSZ), jnp.float32),
        rows=pltpu.VMEM((3, BSZ, H2), jnp.float32),
        zbuf=pltpu.VMEM((CCH, H2), jnp.float32),
        acc=pltpu.VMEM_SHARED((N_PAD, H2), jnp.float32),
        semi0=pltpu.SemaphoreType.DMA,
        semi1=pltpu.SemaphoreType.DMA,
        semi2=pltpu.SemaphoreType.DMA,
        semg0=pltpu.SemaphoreType.DMA,
        semg1=pltpu.SemaphoreType.DMA,
        semg2=pltpu.SemaphoreType.DMA,
        sems0=pltpu.SemaphoreType.DMA,
        sems1=pltpu.SemaphoreType.DMA,
        sems2=pltpu.SemaphoreType.DMA,
    ),
)
def _sc_agg(hwp_hbm, src_hbm, dst_hbm, ew_hbm, out_hbm, srcv, dstv, ewv,
            rows, zbuf, acc, semi0, semi1, semi2, semg0, semg1, semg2,
            sems0, sems1, sems2):
  cid = lax.axis_index("c")
  sid = lax.axis_index("s")
  semi = (semi0, semi1, semi2)
  semg = (semg0, semg1, semg2)
  sems = (sems0, sems1, sems2)
  base = sid * EPT
  table = hwp_hbm.at[cid]

  def idx_start(j, p):
    off = base + j * BSZ
    pltpu.async_copy(src_hbm.at[pl.ds(off, BSZ)], srcv.at[p], semi[p])
    pltpu.async_copy(dst_hbm.at[pl.ds(off, BSZ)], dstv.at[j & 3], semi[p])
    pltpu.async_copy(ew_hbm.at[pl.ds(off, BSZ)], ewv.at[p], semi[p])

  def idx_wait(j, p):
    off = base + j * BSZ
    pltpu.make_async_copy(src_hbm.at[pl.ds(off, BSZ)], srcv.at[p],
                          semi[p]).wait()
    pltpu.make_async_copy(dst_hbm.at[pl.ds(off, BSZ)], dstv.at[j & 3],
                          semi[p]).wait()
    pltpu.make_async_copy(ew_hbm.at[pl.ds(off, BSZ)], ewv.at[p],
                          semi[p]).wait()

  def gather_start(p):
    pltpu.async_copy(table.at[srcv.at[p]], rows.at[p], semg[p])

  def gather_wait(p):
    pltpu.make_async_copy(table.at[srcv.at[p]], rows.at[p], semg[p]).wait()

  def scat_start(j, p):
    pltpu.async_copy(rows.at[p], acc.at[dstv.at[j & 3]], sems[p], add=True)

  def scat_wait(j, p):
    pltpu.make_async_copy(rows.at[p], acc.at[dstv.at[j & 3]],
                          sems[p]).wait()

  def scale(p):
    @plsc.parallel_loop(0, BSZ, unroll=8)
    def _(e):
      ew16 = plsc.load_gather(ewv.at[p], [jnp.full((L,), e, jnp.int32)])
      for k in range(H2 // L):
        rows[p, e, pl.ds(k * L, L)] = rows[p, e, pl.ds(k * L, L)] * ew16

  for j in range(3):
    idx_start(j, j)

  _zero_vmem_2d(zbuf, CCH, H2)
  for k in range(RPT // CCH):
    pltpu.sync_copy(zbuf, acc.at[pl.ds(sid * RPT + k * CCH, CCH)])
  plsc.subcore_barrier()

  idx_wait(0, 0)
  gather_start(0)
  idx_wait(1, 1)
  gather_start(1)

  def step(j, i, first):
    pn = (i + 2) % 3

    @pl.when(jnp.logical_not(first))
    def _():
      scat_wait(j - 1, pn)

    @pl.when(j + 2 < NBATCH)
    def _():
      idx_wait(j + 2, pn)
      gather_start(pn)

    gather_wait(i)
    scale(i)

    @pl.when(j + 3 < NBATCH)
    def _():
      idx_start(j + 3, i)

    scat_start(j, i)

  def tri(t, _):
    j = 3 * t
    step(j, 0, t <= 0)
    step(j + 1, 1, False)
    step(j + 2, 2, False)
    return 0

  lax.fori_loop(0, NTRI, tri, 0)
  step(NBATCH - 1, 0, False)
  scat_wait(NBATCH - 1, 0)
  plsc.subcore_barrier()

  pltpu.sync_copy(acc.at[pl.ds(sid * RPT, RPT)],
                  out_hbm.at[cid, pl.ds(sid * RPT, RPT)])


def _dinv_body(degp_ref, out_ref):
  deg = degp_ref[0] + degp_ref[1] + 2.0
  out_ref[...] = jnp.where(deg > 0, lax.rsqrt(jnp.where(deg > 0, deg, 1.0)),
                           0.0)


def _tc_dinv(degp):
  return pl.pallas_call(
      _dinv_body,
      out_shape=jax.ShapeDtypeStruct((NROW, 128), jnp.float32),
  )(degp)


def _mm_body(x_ref, w_ref, dinv_ref, out_ref):
  r = jnp.dot(x_ref[...], w_ref[...], preferred_element_type=jnp.float32)
  r = r * dinv_ref[...]
  out_ref[0] = r[:, :H2]
  out_ref[1] = r[:, H2:]


def _tc_mm(x, w, dinv):
  d = x.shape[1]
  return pl.pallas_call(
      _mm_body,
      grid=(GRID,),
      in_specs=[
          pl.BlockSpec((RB, d), lambda i: (i, 0)),
          pl.BlockSpec((d, HID), lambda i: (0, 0)),
          pl.BlockSpec((RB, 1), lambda i: (i, 0)),
      ],
      out_specs=pl.BlockSpec((NC, RB, H2), lambda i: (0, i, 0)),
      out_shape=jax.ShapeDtypeStruct((NC, N_PAD, H2), jnp.float32),
  )(x, w, dinv)


def _post_body(agg_ref, hwp_ref, dinv_ref, b_ref, g_ref, be_ref, out_ref, *,
               relu):
  agg = jnp.concatenate([agg_ref[0], agg_ref[1]], axis=1)
  hwp = jnp.concatenate([hwp_ref[0], hwp_ref[1]], axis=1)
  dinv = dinv_ref[...]
  h = dinv * agg + 2.0 * dinv * hwp + b_ref[...]
  mu = jnp.mean(h, axis=-1, keepdims=True)
  var = jnp.mean((h - mu) * (h - mu), axis=-1, keepdims=True)
  h = (h - mu) * lax.rsqrt(var + 1e-5) * g_ref[...] + be_ref[...]
  if relu:
    h = jnp.maximum(h, 0.0)
  out_ref[...] = h


def _tc_post(agg, hwp, dinv, b, g, be, relu):
  return pl.pallas_call(
      functools.partial(_post_body, relu=relu),
      grid=(GRID,),
      in_specs=[
          pl.BlockSpec((NC, RB, H2), lambda i: (0, i, 0)),
          pl.BlockSpec((NC, RB, H2), lambda i: (0, i, 0)),
          pl.BlockSpec((RB, 1), lambda i: (i, 0)),
          pl.BlockSpec((1, HID), lambda i: (0, 0)),
          pl.BlockSpec((1, HID), lambda i: (0, 0)),
          pl.BlockSpec((1, HID), lambda i: (0, 0)),
      ],
      out_specs=pl.BlockSpec((RB, HID), lambda i: (i, 0)),
      out_shape=jax.ShapeDtypeStruct((N_PAD, HID), jnp.float32),
  )(agg, hwp, dinv, b, g, be)


def _postmm_body(agg_ref, hwp_ref, dinv_ref, b_ref, g_ref, be_ref, w_ref,
                 out_ref):
  agg = jnp.concatenate([agg_ref[0], agg_ref[1]], axis=1)
  hwp = jnp.concatenate([hwp_ref[0], hwp_ref[1]], axis=1)
  dinv = dinv_ref[...]
  h = dinv * agg + 2.0 * dinv * hwp + b_ref[...]
  mu = jnp.mean(h, axis=-1, keepdims=True)
  var = jnp.mean((h - mu) * (h - mu), axis=-1, keepdims=True)
  h = (h - mu) * lax.rsqrt(var + 1e-5) * g_ref[...] + be_ref[...]
  h = jnp.maximum(h, 0.0)
  r = jnp.dot(h, w_ref[...], preferred_element_type=jnp.float32) * dinv
  out_ref[0] = r[:, :H2]
  out_ref[1] = r[:, H2:]


def _tc_postmm(agg, hwp, dinv, b, g, be, w):
  return pl.pallas_call(
      _postmm_body,
      grid=(GRID,),
      in_specs=[
          pl.BlockSpec((NC, RB, H2), lambda i: (0, i, 0)),
          pl.BlockSpec((NC, RB, H2), lambda i: (0, i, 0)),
          pl.BlockSpec((RB, 1), lambda i: (i, 0)),
          pl.BlockSpec((1, HID), lambda i: (0, 0)),
          pl.BlockSpec((1, HID), lambda i: (0, 0)),
          pl.BlockSpec((1, HID), lambda i: (0, 0)),
          pl.BlockSpec((HID, HID), lambda i: (0, 0)),
      ],
      out_specs=pl.BlockSpec((NC, RB, H2), lambda i: (0, i, 0)),
      out_shape=jax.ShapeDtypeStruct((NC, N_PAD, H2), jnp.float32),
  )(agg, hwp, dinv, b, g, be, w)


def _tail_body(agg_ref, hwp_ref, dinv_ref, b2_ref, g2_ref, be2_ref, x_ref,
               wih_ref, bih_ref, bhh_ref, wd1_ref, bd1_ref, wd2_ref, bd2_ref,
               wd3_ref, bd3_ref, out_ref):
  agg = jnp.concatenate([agg_ref[0], agg_ref[1]], axis=1)
  hwp = jnp.concatenate([hwp_ref[0], hwp_ref[1]], axis=1)
  dinv = dinv_ref[...]
  h = dinv * agg + 2.0 * dinv * hwp + b2_ref[...]
  mu = jnp.mean(h, axis=-1, keepdims=True)
  var = jnp.mean((h - mu) * (h - mu), axis=-1, keepdims=True)
  h = (h - mu) * lax.rsqrt(var + 1e-5) * g2_ref[...] + be2_ref[...]
  gi = jnp.dot(h, wih_ref[...], preferred_element_type=jnp.float32) + \
      bih_ref[...]
  bhh = bhh_ref[...]
  r = jax.nn.sigmoid(gi[:, :HID] + bhh[:, :HID])
  z = jax.nn.sigmoid(gi[:, HID:2 * HID] + bhh[:, HID:2 * HID])
  n = jnp.tanh(gi[:, 2 * HID:] + r * bhh[:, 2 * HID:])
  temporal = (1.0 - z) * n
  d1 = jnp.dot(temporal, wd1_ref[:HID, :],
               preferred_element_type=jnp.float32)
  d1 = d1 + jnp.dot(x_ref[...], wd1_ref[HID:, :],
                    preferred_element_type=jnp.float32)
  d1 = jnp.maximum(d1 + bd1_ref[...], 0.0)
  d2 = jnp.maximum(
      jnp.dot(d1, wd2_ref[...], preferred_element_type=jnp.float32) +
      bd2_ref[...], 0.0)
  pred = jnp.dot(d2, wd3_ref[...], preferred_element_type=jnp.float32) + \
      bd3_ref[...]
  out_ref[...] = jnp.clip(pred, -5.0, 5.0)


def _tc_tail(agg, hwp, dinv, b2, g2, be2, x, wih_t, b_ih, b_hh, wd1, bd1,
             wd2, bd2, wd3p, bd3p):
  return pl.pallas_call(
      _tail_body,
      grid=(GRID,),
      in_specs=[
          pl.BlockSpec((NC, RB, H2), lambda i: (0, i, 0)),
          pl.BlockSpec((NC, RB, H2), lambda i: (0, i, 0)),
          pl.BlockSpec((RB, 1), lambda i: (i, 0)),
          pl.BlockSpec((1, HID), lambda i: (0, 0)),
          pl.BlockSpec((1, HID), lambda i: (0, 0)),
          pl.BlockSpec((1, HID), lambda i: (0, 0)),
          pl.BlockSpec((RB, IN_DIM), lambda i: (i, 0)),
          pl.BlockSpec((HID, 3 * HID), lambda i: (0, 0)),
          pl.BlockSpec((1, 3 * HID), lambda i: (0, 0)),
          pl.BlockSpec((1, 3 * HID), lambda i: (0, 0)),
          pl.BlockSpec((HID + IN_DIM, HID), lambda i: (0, 0)),
          pl.BlockSpec((1, HID), lambda i: (0, 0)),
          pl.BlockSpec((HID, H2), lambda i: (0, 0)),
          pl.BlockSpec((1, H2), lambda i: (0, 0)),
          pl.BlockSpec((H2, 128), lambda i: (0, 0)),
          pl.BlockSpec((1, 128), lambda i: (0, 0)),
      ],
      out_specs=pl.BlockSpec((RB, 128), lambda i: (i, 0)),
      out_shape=jax.ShapeDtypeStruct((N_PAD, 128), jnp.float32),
  )(agg, hwp, dinv, b2, g2, be2, x, wih_t, b_ih, b_hh, wd1, bd1, wd2, bd2,
    wd3p, bd3p)


@jax.jit
def kernel(x, edge_index, edge_weight, W1, b1, g1, be1, W2, b2, g2, be2,
           W_ih, W_hh, b_ih, b_hh, Wd1, bd1, Wd2, bd2, Wd3, bd3):
  src = edge_index[0]
  dst = edge_index[1]
  x_pad = jnp.pad(x, ((0, N_PAD - N), (0, 0)))

  degp = _sc_deg(dst, edge_weight)
  dinv = _tc_dinv(degp).reshape(N_PAD, 1)

  hwp1 = _tc_mm(x_pad, W1, dinv)
  agg1 = _sc_agg(hwp1, src, dst, edge_weight)
  hwp2 = _tc_postmm(agg1, hwp1, dinv, b1.reshape(1, HID),
                    g1.reshape(1, HID), be1.reshape(1, HID), W2)
  agg2 = _sc_agg(hwp2, src, dst, edge_weight)

  pred_pad = _tc_tail(
      agg2, hwp2, dinv, b2.reshape(1, HID), g2.reshape(1, HID),
      be2.reshape(1, HID), x_pad, W_ih.T, b_ih.reshape(1, 3 * HID),
      b_hh.reshape(1, 3 * HID), Wd1, bd1.reshape(1, HID), Wd2,
      bd2.reshape(1, H2),
      jnp.pad(Wd3, ((0, 0), (0, 128 - Wd3.shape[1]))),
      jnp.pad(bd3, (0, 128 - bd3.shape[0])).reshape(1, 128))
  return pred_pad[:N, :2]


# speedup vs baseline: 1.1367x; 1.0407x over previous
import functools

import jax
import jax.numpy as jnp
from jax import lax
from jax.experimental import pallas as pl
from jax.experimental.pallas import tpu as pltpu
from jax.experimental.pallas import tpu_sc as plsc

N = 10000
N_PAD = 10240
NROW = N_PAD // 128
E = 320000
IN_DIM = 128
HID = 256
H2 = HID // 2
NC, NS, L = 2, 16, 16
RB = 512
GRID = N_PAD // RB

EPT = E // NS
BSZ = 80
NBATCH = EPT // BSZ
RPT = N_PAD // NS
CCH = 32

EPW = E // (NC * NS)
DBS = 400
NDB = EPW // DBS

_mesh = plsc.VectorSubcoreMesh(
    core_axis_name="c", subcore_axis_name="s", num_cores=NC, num_subcores=NS)

_sc_params = pltpu.CompilerParams(use_tc_tiling_on_sc=False,
                                 needs_layout_passes=False)


def _zero_vmem_2d(ref, rows, cols):
  def body(r, _):
    for k in range(cols // L):
      ref[r, pl.ds(k * L, L)] = jnp.zeros((L,), jnp.float32)
    return 0
  lax.fori_loop(0, rows, body, 0)


def _iota_rows(ref, n):
  base = lax.iota(jnp.int32, L)
  for j in range(n // L):
    ref[pl.ds(j * L, L)] = base + (j * L)


@functools.partial(
    pl.kernel,
    out_type=jax.ShapeDtypeStruct((NC, NROW, 128), jnp.float32),
    mesh=_mesh,
    compiler_params=_sc_params,
    scratch_types=dict(
        dstv=pltpu.VMEM((EPW,), jnp.int32),
        ewv=pltpu.VMEM((EPW,), jnp.float32),
        deg_local=pltpu.VMEM((NROW, 128), jnp.float32),
        rowidx=pltpu.VMEM((NROW,), jnp.int32),
        zrow=pltpu.VMEM((NROW // NS, 128), jnp.float32),
        tmp=pltpu.VMEM((NROW // NS, 128), jnp.float32),
        acc=pltpu.VMEM_SHARED((NROW, 128), jnp.float32),
    ),
)
def _sc_deg(dst_hbm, ew_hbm, out_hbm, dstv, ewv, deg_local, rowidx, zrow,
            tmp, acc):
  cid = lax.axis_index("c")
  sid = lax.axis_index("s")
  rpt = NROW // NS

  _zero_vmem_2d(deg_local, NROW, 128)
  _zero_vmem_2d(zrow, rpt, 128)
  _iota_rows(rowidx, NROW)
  pltpu.sync_copy(zrow, acc.at[pl.ds(sid * rpt, rpt)])

  base = cid * (E // NC) + sid * EPW
  pltpu.sync_copy(dst_hbm.at[pl.ds(base, EPW)], dstv)
  pltpu.sync_copy(ew_hbm.at[pl.ds(base, EPW)], ewv)

  @plsc.parallel_loop(0, EPW // L, unroll=4)
  def _(k):
    idx = dstv[pl.ds(k * L, L)]
    w = ewv[pl.ds(k * L, L)]
    plsc.addupdate_scatter(deg_local, [idx >> 7, idx & 127], w)

  plsc.subcore_barrier()
  pltpu.sync_copy(deg_local, acc.at[rowidx], add=True)
  plsc.subcore_barrier()

  pltpu.sync_copy(acc.at[pl.ds(sid * rpt, rpt)],
                  out_hbm.at[cid, pl.ds(sid * rpt, rpt)])


NTRI = (NBATCH - 1) // 3


@functools.partial(
    pl.kernel,
    out_type=jax.ShapeDtypeStruct((NC, N_PAD, H2), jnp.float32),
    mesh=_mesh,
    compiler_params=_sc_params,
    scratch_types=dict(
        srcv=pltpu.VMEM((3, BSZ), jnp.int32),
        dstv=pltpu.VMEM((4, BSZ), jnp.int32),
        ewv=pltpu.VMEM((3, BSZ), jnp.float32),
        rows=pltpu.VMEM((3, BSZ, H2), jnp.float32),
        zbuf=pltpu.VMEM((CCH, H2), jnp.float32),
        acc=pltpu.VMEM_SHARED((N_PAD, H2), jnp.float32),
        semi0=pltpu.SemaphoreType.DMA,
        semi1=pltpu.SemaphoreType.DMA,
        semi2=pltpu.SemaphoreType.DMA,
        semg0=pltpu.SemaphoreType.DMA,
        semg1=pltpu.SemaphoreType.DMA,
        semg2=pltpu.SemaphoreType.DMA,
        sems0=pltpu.SemaphoreType.DMA,
        sems1=pltpu.SemaphoreType.DMA,
        sems2=pltpu.SemaphoreType.DMA,
    ),
)
def _sc_agg(hwp_hbm, src_hbm, dst_hbm, ew_hbm, out_hbm, srcv, dstv, ewv,
            rows, zbuf, acc, semi0, semi1, semi2, semg0, semg1, semg2,
            sems0, sems1, sems2):
  cid = lax.axis_index("c")
  sid = lax.axis_index("s")
  semi = (semi0, semi1, semi2)
  semg = (semg0, semg1, semg2)
  sems = (sems0, sems1, sems2)
  base = sid * EPT
  table = hwp_hbm.at[cid]

  def idx_start(j, p):
    off = base + j * BSZ
    pltpu.async_copy(src_hbm.at[pl.ds(off, BSZ)], srcv.at[p], semi[p])
    pltpu.async_copy(dst_hbm.at[pl.ds(off, BSZ)], dstv.at[j & 3], semi[p])
    pltpu.async_copy(ew_hbm.at[pl.ds(off, BSZ)], ewv.at[p], semi[p])

  def idx_wait(j, p):
    off = base + j * BSZ
    pltpu.make_async_copy(src_hbm.at[pl.ds(off, BSZ)], srcv.at[p],
                          semi[p]).wait()
    pltpu.make_async_copy(dst_hbm.at[pl.ds(off, BSZ)], dstv.at[j & 3],
                          semi[p]).wait()
    pltpu.make_async_copy(ew_hbm.at[pl.ds(off, BSZ)], ewv.at[p],
                          semi[p]).wait()

  def gather_start(p):
    pltpu.async_copy(table.at[srcv.at[p]], rows.at[p], semg[p])

  def gather_wait(p):
    pltpu.make_async_copy(table.at[srcv.at[p]], rows.at[p], semg[p]).wait()

  def scat_start(j, p):
    pltpu.async_copy(rows.at[p], acc.at[dstv.at[j & 3]], sems[p], add=True)

  def scat_wait(j, p):
    pltpu.make_async_copy(rows.at[p], acc.at[dstv.at[j & 3]],
                          sems[p]).wait()

  def scale(p):
    @plsc.parallel_loop(0, BSZ, unroll=8)
    def _(e):
      ew16 = plsc.load_gather(ewv.at[p], [jnp.full((L,), e, jnp.int32)])
      for k in range(H2 // L):
        rows[p, e, pl.ds(k * L, L)] = rows[p, e, pl.ds(k * L, L)] * ew16

  for j in range(3):
    idx_start(j, j)

  _zero_vmem_2d(zbuf, CCH, H2)
  for k in range(RPT // CCH):
    pltpu.sync_copy(zbuf, acc.at[pl.ds(sid * RPT + k * CCH, CCH)])
  plsc.subcore_barrier()

  idx_wait(0, 0)
  gather_start(0)
  idx_wait(1, 1)
  gather_start(1)

  def step(j, i, first):
    pn = (i + 2) % 3

    @pl.when(jnp.logical_not(first))
    def _():
      scat_wait(j - 1, pn)

    @pl.when(j + 2 < NBATCH)
    def _():
      idx_wait(j + 2, pn)
      gather_start(pn)

    gather_wait(i)
    scale(i)

    @pl.when(j + 3 < NBATCH)
    def _():
      idx_start(j + 3, i)

    scat_start(j, i)

  def tri(t, _):
    j = 3 * t
    step(j, 0, t <= 0)
    step(j + 1, 1, False)
    step(j + 2, 2, False)
    return 0

  lax.fori_loop(0, NTRI, tri, 0)
  step(NBATCH - 1, 0, False)
  scat_wait(NBATCH - 1, 0)
  plsc.subcore_barrier()

  pltpu.sync_copy(acc.at[pl.ds(sid * RPT, RPT)],
                  out_hbm.at[cid, pl.ds(sid * RPT, RPT)])


def _dinv_body(degp_ref, out_ref):
  deg = degp_ref[0] + degp_ref[1] + 2.0
  out_ref[...] = jnp.where(deg > 0, lax.rsqrt(jnp.where(deg > 0, deg, 1.0)),
                           0.0)


def _tc_dinv(degp):
  return pl.pallas_call(
      _dinv_body,
      out_shape=jax.ShapeDtypeStruct((NROW, 128), jnp.float32),
  )(degp)


def _mm_body(x_ref, w_ref, dinv_ref, out_ref):
  r = jnp.dot(x_ref[...], w_ref[...], preferred_element_type=jnp.float32)
  r = r * dinv_ref[...]
  out_ref[0] = r[:, :H2]
  out_ref[1] = r[:, H2:]


def _tc_mm(x, w, dinv):
  d = x.shape[1]
  return pl.pallas_call(
      _mm_body,
      grid=(GRID,),
      in_specs=[
          pl.BlockSpec((RB, d), lambda i: (i, 0)),
          pl.BlockSpec((d, HID), lambda i: (0, 0)),
          pl.BlockSpec((RB, 1), lambda i: (i, 0)),
      ],
      out_specs=pl.BlockSpec((NC, RB, H2), lambda i: (0, i, 0)),
      out_shape=jax.ShapeDtypeStruct((NC, N_PAD, H2), jnp.float32),
  )(x, w, dinv)


def _post_body(agg_ref, hwp_ref, dinv_ref, b_ref, g_ref, be_ref, out_ref, *,
               relu):
  agg = jnp.concatenate([agg_ref[0], agg_ref[1]], axis=1)
  hwp = jnp.concatenate([hwp_ref[0], hwp_ref[1]], axis=1)
  dinv = dinv_ref[...]
  h = dinv * agg + 2.0 * dinv * hwp + b_ref[...]
  mu = jnp.mean(h, axis=-1, keepdims=True)
  var = jnp.mean((h - mu) * (h - mu), axis=-1, keepdims=True)
  h = (h - mu) * lax.rsqrt(var + 1e-5) * g_ref[...] + be_ref[...]
  if relu:
    h = jnp.maximum(h, 0.0)
  out_ref[...] = h


def _tc_post(agg, hwp, dinv, b, g, be, relu):
  return pl.pallas_call(
      functools.partial(_post_body, relu=relu),
      grid=(GRID,),
      in_specs=[
          pl.BlockSpec((NC, RB, H2), lambda i: (0, i, 0)),
          pl.BlockSpec((NC, RB, H2), lambda i: (0, i, 0)),
          pl.BlockSpec((RB, 1), lambda i: (i, 0)),
          pl.BlockSpec((1, HID), lambda i: (0, 0)),
          pl.BlockSpec((1, HID), lambda i: (0, 0)),
          pl.BlockSpec((1, HID), lambda i: (0, 0)),
      ],
      out_specs=pl.BlockSpec((RB, HID), lambda i: (i, 0)),
      out_shape=jax.ShapeDtypeStruct((N_PAD, HID), jnp.float32),
  )(agg, hwp, dinv, b, g, be)


def _postmm_body(agg_ref, hwp_ref, dinv_ref, b_ref, g_ref, be_ref, w_ref,
                 out_ref):
  agg = jnp.concatenate([agg_ref[0], agg_ref[1]], axis=1)
  hwp = jnp.concatenate([hwp_ref[0], hwp_ref[1]], axis=1)
  dinv = dinv_ref[...]
  h = dinv * agg + 2.0 * dinv * hwp + b_ref[...]
  mu = jnp.mean(h, axis=-1, keepdims=True)
  var = jnp.mean((h - mu) * (h - mu), axis=-1, keepdims=True)
  h = (h - mu) * lax.rsqrt(var + 1e-5) * g_ref[...] + be_ref[...]
  h = jnp.maximum(h, 0.0)
  r = jnp.dot(h, w_ref[...], preferred_element_type=jnp.float32) * dinv
  out_ref[0] = r[:, :H2]
  out_ref[1] = r[:, H2:]


def _tc_postmm(agg, hwp, dinv, b, g, be, w):
  return pl.pallas_call(
      _postmm_body,
      grid=(GRID,),
      in_specs=[
          pl.BlockSpec((NC, RB, H2), lambda i: (0, i, 0)),
          pl.BlockSpec((NC, RB, H2), lambda i: (0, i, 0)),
          pl.BlockSpec((RB, 1), lambda i: (i, 0)),
          pl.BlockSpec((1, HID), lambda i: (0, 0)),
          pl.BlockSpec((1, HID), lambda i: (0, 0)),
          pl.BlockSpec((1, HID), lambda i: (0, 0)),
          pl.BlockSpec((HID, HID), lambda i: (0, 0)),
      ],
      out_specs=pl.BlockSpec((NC, RB, H2), lambda i: (0, i, 0)),
      out_shape=jax.ShapeDtypeStruct((NC, N_PAD, H2), jnp.float32),
  )(agg, hwp, dinv, b, g, be, w)


def _tail_body(agg_ref, hwp_ref, dinv_ref, b2_ref, g2_ref, be2_ref, x_ref,
               wih_ref, bih_ref, bhh_ref, wd1_ref, bd1_ref, wd2_ref, bd2_ref,
               wd3_ref, bd3_ref, out_ref):
  agg = jnp.concatenate([agg_ref[0], agg_ref[1]], axis=1)
  hwp = jnp.concatenate([hwp_ref[0], hwp_ref[1]], axis=1)
  dinv = dinv_ref[...]
  h = dinv * agg + 2.0 * dinv * hwp + b2_ref[...]
  mu = jnp.mean(h, axis=-1, keepdims=True)
  var = jnp.mean((h - mu) * (h - mu), axis=-1, keepdims=True)
  h = (h - mu) * lax.rsqrt(var + 1e-5) * g2_ref[...] + be2_ref[...]
  gi = jnp.dot(h, wih_ref[...], preferred_element_type=jnp.float32) + \
      bih_ref[...]
  bhh = bhh_ref[...]
  r = jax.nn.sigmoid(gi[:, :HID] + bhh[:, :HID])
  z = jax.nn.sigmoid(gi[:, HID:2 * HID] + bhh[:, HID:2 * HID])
  n = jnp.tanh(gi[:, 2 * HID:] + r * bhh[:, 2 * HID:])
  temporal = (1.0 - z) * n
  d1 = jnp.dot(temporal, wd1_ref[:HID, :],
               preferred_element_type=jnp.float32)
  d1 = d1 + jnp.dot(x_ref[...], wd1_ref[HID:, :],
                    preferred_element_type=jnp.float32)
  d1 = jnp.maximum(d1 + bd1_ref[...], 0.0)
  d2 = jnp.maximum(
      jnp.dot(d1, wd2_ref[...], preferred_element_type=jnp.float32) +
      bd2_ref[...], 0.0)
  pred = jnp.dot(d2, wd3_ref[...], preferred_element_type=jnp.float32) + \
      bd3_ref[...]
  out_ref[...] = jnp.clip(pred, -5.0, 5.0)


def _tc_tail(agg, hwp, dinv, b2, g2, be2, x, wih_t, b_ih, b_hh, wd1, bd1,
             wd2, bd2, wd3p, bd3p):
  return pl.pallas_call(
      _tail_body,
      grid=(GRID,),
      in_specs=[
          pl.BlockSpec((NC, RB, H2), lambda i: (0, i, 0)),
          pl.BlockSpec((NC, RB, H2), lambda i: (0, i, 0)),
          pl.BlockSpec((RB, 1), lambda i: (i, 0)),
          pl.BlockSpec((1, HID), lambda i: (0, 0)),
          pl.BlockSpec((1, HID), lambda i: (0, 0)),
          pl.BlockSpec((1, HID), lambda i: (0, 0)),
          pl.BlockSpec((RB, IN_DIM), lambda i: (i, 0)),
          pl.BlockSpec((HID, 3 * HID), lambda i: (0, 0)),
          pl.BlockSpec((1, 3 * HID), lambda i: (0, 0)),
          pl.BlockSpec((1, 3 * HID), lambda i: (0, 0)),
          pl.BlockSpec((HID + IN_DIM, HID), lambda i: (0, 0)),
          pl.BlockSpec((1, HID), lambda i: (0, 0)),
          pl.BlockSpec((HID, H2), lambda i: (0, 0)),
          pl.BlockSpec((1, H2), lambda i: (0, 0)),
          pl.BlockSpec((H2, 128), lambda i: (0, 0)),
          pl.BlockSpec((1, 128), lambda i: (0, 0)),
      ],
      out_specs=pl.BlockSpec((RB, 128), lambda i: (i, 0)),
      out_shape=jax.ShapeDtypeStruct((N_PAD, 128), jnp.float32),
  )(agg, hwp, dinv, b2, g2, be2, x, wih_t, b_ih, b_hh, wd1, bd1, wd2, bd2,
    wd3p, bd3p)


@jax.jit
def kernel(x, edge_index, edge_weight, W1, b1, g1, be1, W2, b2, g2, be2,
           W_ih, W_hh, b_ih, b_hh, Wd1, bd1, Wd2, bd2, Wd3, bd3):
  src = edge_index[0]
  dst = edge_index[1]
  x_pad = jnp.pad(x, ((0, N_PAD - N), (0, 0)))

  degp = _sc_deg(dst, edge_weight)
  dinv = _tc_dinv(degp).reshape(N_PAD, 1)

  hwp1 = _tc_mm(x_pad, W1, dinv)
  agg1 = _sc_agg(hwp1, src, dst, edge_weight)
  hwp2 = _tc_postmm(agg1, hwp1, dinv, b1.reshape(1, HID),
                    g1.reshape(1, HID), be1.reshape(1, HID), W2)
  agg2 = _sc_agg(hwp2, src, dst, edge_weight)

  pred_pad = _tc_tail(
      agg2, hwp2, dinv, b2.reshape(1, HID), g2.reshape(1, HID),
      be2.reshape(1, HID), x_pad, W_ih.T, b_ih.reshape(1, 3 * HID),
      b_hh.reshape(1, 3 * HID), Wd1, bd1.reshape(1, HID), Wd2,
      bd2.reshape(1, H2),
      jnp.pad(Wd3, ((0, 0), (0, 128 - Wd3.shape[1]))),
      jnp.pad(bd3, (0, 128 - bd3.shape[0])).reshape(1, 128))
  return pred_pad[:N, :2]
